# baseline (speedup 1.0000x reference)
import functools

import jax
import jax.numpy as jnp
from jax import lax
from jax.experimental import pallas as pl
from jax.experimental.pallas import tpu as pltpu
from jax.experimental.pallas import tpu_sc as plsc

N = 10000
E = 320000
D = 128
NUM_LAYERS = 5

NC = 2
NS = 16
NW = NC * NS
EPW = E // NW
CHUNK = 40
EPW_PAD = 10240
NCHUNK = EPW_PAD // CHUNK
PAD = EPW_PAD - EPW
RR = 4
IR = 8
NPAD = 10240
ROWS_PER_TILE = NPAD // NS


def _sc_segment_sum(h, src, dst, zrows):
    mesh = plsc.VectorSubcoreMesh(core_axis_name="c", subcore_axis_name="s")

    @functools.partial(
        pl.kernel,
        mesh=mesh,
        out_type=jax.ShapeDtypeStruct((NC, NPAD, D), jnp.float32),
        scratch_types=(
            [pltpu.VMEM((CHUNK,), jnp.int32) for _ in range(IR)]
            + [pltpu.VMEM((CHUNK,), jnp.int32) for _ in range(IR)]
            + [pltpu.VMEM((CHUNK, D), jnp.float32) for _ in range(RR)]
            + [pltpu.VMEM_SHARED((NPAD, D), jnp.float32)]
            + [pltpu.SemaphoreType.DMA for _ in range(IR + 2 * RR + 1)]
        ),
    )
    def seg_sum(h_hbm, src_hbm, dst_hbm, z_hbm, out_hbm, *scr):
        sidx = scr[:IR]
        didx = scr[IR:2 * IR]
        rows = scr[2 * IR:2 * IR + RR]
        acc = scr[2 * IR + RR]
        sems = scr[2 * IR + RR + 1:]
        isem = sems[:IR]
        gsem = sems[IR:IR + RR]
        ssem = sems[IR + RR:IR + 2 * RR]
        zsem = sems[IR + 2 * RR]
        cid = lax.axis_index("c")
        sid = lax.axis_index("s")
        wid = cid * NS + sid
        base = wid * EPW_PAD

        def fire_idx(c, i):
            off = base + c * CHUNK
            pltpu.async_copy(src_hbm.at[pl.ds(off, CHUNK)], sidx[i], isem[i])
            pltpu.async_copy(dst_hbm.at[pl.ds(off, CHUNK)], didx[i], isem[i])

        def wait_idx(i):
            pltpu.make_async_copy(
                src_hbm.at[pl.ds(base, CHUNK)], sidx[i], isem[i]).wait()
            pltpu.make_async_copy(
                dst_hbm.at[pl.ds(base, CHUNK)], didx[i], isem[i]).wait()

        zcp = pltpu.make_async_copy(
            z_hbm, acc.at[pl.ds(sid * ROWS_PER_TILE, ROWS_PER_TILE)], zsem)
        zcp.start()
        for c in range(3):
            fire_idx(c, c)
        for b in range(2):
            wait_idx(b)
            pltpu.async_copy(h_hbm.at[sidx[b]], rows[b], gsem[b])
        zcp.wait()
        plsc.subcore_barrier()

        def ring(k, carry):
            for u in range(IR):
                j = IR * k + u
                b = u % RR
                b2 = (u + 2) % RR
                i2 = (u + 2) % IR
                i3 = (u + 3) % IR

                @pl.when(j < NCHUNK)
                def _process():
                    pltpu.make_async_copy(
                        h_hbm.at[sidx[u]], rows[b], gsem[b]).wait()
                    pltpu.async_copy(
                        rows[b], acc.at[didx[u]], ssem[b], add=True)

                    @pl.when(j >= 2)
                    def _drain_s():
                        pltpu.make_async_copy(
                            rows[b2], acc.at[didx[i2]], ssem[b2]).wait()

                    @pl.when(j + 2 < NCHUNK)
                    def _fire_g():
                        wait_idx(i2)
                        pltpu.async_copy(
                            h_hbm.at[sidx[i2]], rows[b2], gsem[b2])

                    @pl.when(j + 3 < NCHUNK)
                    def _fire_i():
                        fire_idx(j + 3, i3)
            return carry
        lax.fori_loop(0, (NCHUNK + IR - 1) // IR, ring, 0)

        for c in (NCHUNK - 2, NCHUNK - 1):
            pltpu.make_async_copy(
                rows[c % RR], acc.at[didx[c % IR]], ssem[c % RR]).wait()
        plsc.subcore_barrier()

        pltpu.sync_copy(
            acc.at[pl.ds(sid * ROWS_PER_TILE, ROWS_PER_TILE)],
            out_hbm.at[cid, pl.ds(sid * ROWS_PER_TILE, ROWS_PER_TILE)])

    return seg_sum(h, src, dst, zrows)


def _dense_body(eps_ref, p2_ref, h_ref, w1_ref, b1_ref, g1_ref, be1_ref,
                w2_ref, b2_ref, g2_ref, be2_ref, hout_ref, sum_ref):
    hcur = h_ref[...]
    p2 = p2_ref[:, :N, :]
    pooled = p2[0] + p2[1] + (1.0 + eps_ref[0, 0]) * hcur
    z = jnp.dot(pooled, w1_ref[...], preferred_element_type=jnp.float32)
    z = z + b1_ref[...]
    mu = jnp.mean(z, axis=0, keepdims=True)
    var = jnp.mean((z - mu) ** 2, axis=0, keepdims=True)
    z = g1_ref[...] * (z - mu) * lax.rsqrt(var + 1e-5) + be1_ref[...]
    z = jnp.maximum(z, 0.0)
    z = jnp.dot(z, w2_ref[...], preferred_element_type=jnp.float32)
    z = z + b2_ref[...]
    mu2 = jnp.mean(z, axis=0, keepdims=True)
    var2 = jnp.mean((z - mu2) ** 2, axis=0, keepdims=True)
    z = g2_ref[...] * (z - mu2) * lax.rsqrt(var2 + 1e-5) + be2_ref[...]
    hn = jnp.maximum(z, 0.0)
    hout_ref[...] = hn
    sum_ref[...] = jnp.sum(hn, axis=0, keepdims=True)


def _dense_layer(parts, h, eps11, p, interpret=False):
    return pl.pallas_call(
        _dense_body,
        out_shape=(jax.ShapeDtypeStruct((N, D), jnp.float32),
                   jax.ShapeDtypeStruct((1, D), jnp.float32)),
        interpret=interpret,
    )(eps11, parts, h,
      p["W1"], p["b1"].reshape(1, D), p["g1"].reshape(1, D),
      p["be1"].reshape(1, D),
      p["W2"], p["b2"].reshape(1, D), p["g2"].reshape(1, D),
      p["be2"].reshape(1, D))


def _heads_body(x_ref, sums_ref, ws_ref, bs_ref, out_ref):
    acc = jnp.sum(bs_ref[...], axis=0)
    xsum = jnp.sum(x_ref[...], axis=0, keepdims=True)
    for l in range(NUM_LAYERS):
        s = xsum if l == 0 else sums_ref[pl.ds(l - 1, 1), :]
        acc = acc + jnp.dot(s, ws_ref[l],
                            preferred_element_type=jnp.float32)
    out_ref[...] = acc


def _heads(x, sums4, ws, bs, interpret=False):
    return pl.pallas_call(
        _heads_body,
        out_shape=jax.ShapeDtypeStruct((1, D), jnp.float32),
        interpret=interpret,
    )(x, sums4, ws, bs)


def kernel(x, edge_index, params):
    srcw = edge_index[0].reshape(NW, EPW)
    dstw = edge_index[1].reshape(NW, EPW)
    spad = jnp.zeros((NW, PAD), jnp.int32)
    dpad = jnp.broadcast_to(N + jnp.arange(PAD, dtype=jnp.int32), (NW, PAD))
    src = jnp.concatenate([srcw, spad], axis=1).reshape(-1)
    dst = jnp.concatenate([dstw, dpad], axis=1).reshape(-1)
    zrows = jnp.zeros((ROWS_PER_TILE, D), jnp.float32)
    h = x
    sums = []
    for l in range(NUM_LAYERS - 1):
        parts = _sc_segment_sum(h, src, dst, zrows)
        h, s = _dense_layer(parts, h, params["eps"][l].reshape(1, 1),
                            params["layers"][l])
        sums.append(s)
    ws = jnp.stack([params["pred"][l]["W"] for l in range(NUM_LAYERS)])
    bs = jnp.stack([params["pred"][l]["b"].reshape(1, D)
                    for l in range(NUM_LAYERS)])
    sums4 = jnp.concatenate(sums, axis=0)
    return _heads(x, sums4, ws, bs)


# speedup vs baseline: 2.7485x; 2.7485x over previous
import functools

import jax
import jax.numpy as jnp
from jax import lax
from jax.experimental import pallas as pl
from jax.experimental.pallas import tpu as pltpu
from jax.experimental.pallas import tpu_sc as plsc

N = 10000
E = 320000
D = 128
NUM_LAYERS = 5

NC = 2
NS = 16
NW = NC * NS
EPW = E // NW
CHUNK = 40
NCHUNK = EPW // CHUNK
RR = 6
IR = 12
NPAD = 10240
ROWS_PER_TILE = NPAD // NS


def _sc_segment_sum(h, src, dst, zrows):
    mesh = plsc.VectorSubcoreMesh(core_axis_name="c", subcore_axis_name="s")

    @functools.partial(
        pl.kernel,
        mesh=mesh,
        out_type=jax.ShapeDtypeStruct((NC, NPAD, D), jnp.float32),
        scratch_types=(
            [pltpu.VMEM((CHUNK,), jnp.int32) for _ in range(IR)]
            + [pltpu.VMEM((CHUNK,), jnp.int32) for _ in range(IR)]
            + [pltpu.VMEM((CHUNK, D), jnp.float32) for _ in range(RR)]
            + [pltpu.VMEM_SHARED((NPAD, D), jnp.float32)]
            + [pltpu.SemaphoreType.DMA for _ in range(IR + 2 * RR + 1)]
        ),
    )
    def seg_sum(h_hbm, src_hbm, dst_hbm, z_hbm, out_hbm, *scr):
        sidx = scr[:IR]
        didx = scr[IR:2 * IR]
        rows = scr[2 * IR:2 * IR + RR]
        acc = scr[2 * IR + RR]
        sems = scr[2 * IR + RR + 1:]
        isem = sems[:IR]
        gsem = sems[IR:IR + RR]
        ssem = sems[IR + RR:IR + 2 * RR]
        zsem = sems[IR + 2 * RR]
        cid = lax.axis_index("c")
        sid = lax.axis_index("s")
        wid = cid * NS + sid
        base = wid * EPW

        def fire_idx(c, i):
            off = base + c * CHUNK
            pltpu.async_copy(src_hbm.at[pl.ds(off, CHUNK)], sidx[i], isem[i])
            pltpu.async_copy(dst_hbm.at[pl.ds(off, CHUNK)], didx[i], isem[i])

        def wait_idx(i):
            pltpu.make_async_copy(
                src_hbm.at[pl.ds(base, CHUNK)], sidx[i], isem[i]).wait()
            pltpu.make_async_copy(
                dst_hbm.at[pl.ds(base, CHUNK)], didx[i], isem[i]).wait()

        zcp = pltpu.make_async_copy(
            z_hbm, acc.at[pl.ds(sid * ROWS_PER_TILE, ROWS_PER_TILE)], zsem)
        zcp.start()
        for c in range(3):
            fire_idx(c, c)
        for b in range(2):
            wait_idx(b)
            pltpu.async_copy(h_hbm.at[sidx[b]], rows[b], gsem[b])
        zcp.wait()
        plsc.subcore_barrier()

        def ring(k, carry):
            for u in range(IR):
                j = IR * k + u
                b = u % RR
                b2 = (u + 2) % RR
                i2 = (u + 2) % IR
                i3 = (u + 3) % IR

                @pl.when(j < NCHUNK)
                def _process():
                    pltpu.make_async_copy(
                        h_hbm.at[sidx[u]], rows[b], gsem[b]).wait()
                    pltpu.async_copy(
                        rows[b], acc.at[didx[u]], ssem[b], add=True)

                    @pl.when(j >= RR - 2)
                    def _drain_s():
                        pltpu.make_async_copy(
                            rows[b2], acc.at[didx[i2]], ssem[b2]).wait()

                    @pl.when(j + 2 < NCHUNK)
                    def _fire_g():
                        wait_idx(i2)
                        pltpu.async_copy(
                            h_hbm.at[sidx[i2]], rows[b2], gsem[b2])

                    @pl.when(j + 3 < NCHUNK)
                    def _fire_i():
                        fire_idx(j + 3, i3)
            return carry
        lax.fori_loop(0, (NCHUNK + IR - 1) // IR, ring, 0)

        for c in range(NCHUNK - (RR - 2), NCHUNK):
            pltpu.make_async_copy(
                rows[c % RR], acc.at[didx[c % IR]], ssem[c % RR]).wait()
        plsc.subcore_barrier()

        pltpu.sync_copy(
            acc.at[pl.ds(sid * ROWS_PER_TILE, ROWS_PER_TILE)],
            out_hbm.at[cid, pl.ds(sid * ROWS_PER_TILE, ROWS_PER_TILE)])

    return seg_sum(h, src, dst, zrows)


def _dense_body(eps_ref, p2_ref, h_ref, w1_ref, b1_ref, g1_ref, be1_ref,
                w2_ref, b2_ref, g2_ref, be2_ref, hout_ref, sum_ref):
    hcur = h_ref[...]
    p2 = p2_ref[:, :N, :]
    pooled = p2[0] + p2[1] + (1.0 + eps_ref[0, 0]) * hcur
    z = jnp.dot(pooled, w1_ref[...], preferred_element_type=jnp.float32)
    z = z + b1_ref[...]
    mu = jnp.mean(z, axis=0, keepdims=True)
    var = jnp.mean((z - mu) ** 2, axis=0, keepdims=True)
    z = g1_ref[...] * (z - mu) * lax.rsqrt(var + 1e-5) + be1_ref[...]
    z = jnp.maximum(z, 0.0)
    z = jnp.dot(z, w2_ref[...], preferred_element_type=jnp.float32)
    z = z + b2_ref[...]
    mu2 = jnp.mean(z, axis=0, keepdims=True)
    var2 = jnp.mean((z - mu2) ** 2, axis=0, keepdims=True)
    z = g2_ref[...] * (z - mu2) * lax.rsqrt(var2 + 1e-5) + be2_ref[...]
    hn = jnp.maximum(z, 0.0)
    hout_ref[...] = hn
    sum_ref[...] = jnp.sum(hn, axis=0, keepdims=True)


def _dense_layer(parts, h, eps11, p, interpret=False):
    return pl.pallas_call(
        _dense_body,
        out_shape=(jax.ShapeDtypeStruct((N, D), jnp.float32),
                   jax.ShapeDtypeStruct((1, D), jnp.float32)),
        interpret=interpret,
    )(eps11, parts, h,
      p["W1"], p["b1"].reshape(1, D), p["g1"].reshape(1, D),
      p["be1"].reshape(1, D),
      p["W2"], p["b2"].reshape(1, D), p["g2"].reshape(1, D),
      p["be2"].reshape(1, D))


def _heads_body(x_ref, sums_ref, ws_ref, bs_ref, out_ref):
    acc = jnp.sum(bs_ref[...], axis=0)
    xsum = jnp.sum(x_ref[...], axis=0, keepdims=True)
    for l in range(NUM_LAYERS):
        s = xsum if l == 0 else sums_ref[pl.ds(l - 1, 1), :]
        acc = acc + jnp.dot(s, ws_ref[l],
                            preferred_element_type=jnp.float32)
    out_ref[...] = acc


def _heads(x, sums4, ws, bs, interpret=False):
    return pl.pallas_call(
        _heads_body,
        out_shape=jax.ShapeDtypeStruct((1, D), jnp.float32),
        interpret=interpret,
    )(x, sums4, ws, bs)


def kernel(x, edge_index, params):
    src = edge_index[0]
    dst = edge_index[1]
    zrows = jnp.zeros((ROWS_PER_TILE, D), jnp.float32)
    h = x
    sums = []
    for l in range(NUM_LAYERS - 1):
        parts = _sc_segment_sum(h, src, dst, zrows)
        h, s = _dense_layer(parts, h, params["eps"][l].reshape(1, 1),
                            params["layers"][l])
        sums.append(s)
    ws = jnp.stack([params["pred"][l]["W"] for l in range(NUM_LAYERS)])
    bs = jnp.stack([params["pred"][l]["b"].reshape(1, D)
                    for l in range(NUM_LAYERS)])
    sums4 = jnp.concatenate(sums, axis=0)
    return _heads(x, sums4, ws, bs)


# speedup vs baseline: 2.7508x; 1.0008x over previous
import functools

import jax
import jax.numpy as jnp
from jax import lax
from jax.experimental import pallas as pl
from jax.experimental.pallas import tpu as pltpu
from jax.experimental.pallas import tpu_sc as plsc

N = 10000
E = 320000
D = 128
NUM_LAYERS = 5

NC = 2
NS = 16
NW = NC * NS
EPW = E // NW
CHUNK = 40
NCHUNK = EPW // CHUNK
RR = 4
IR = 8
NPAD = 10240
ROWS_PER_TILE = NPAD // NS


def _sc_segment_sum(h, src, dst, zrows):
    mesh = plsc.VectorSubcoreMesh(core_axis_name="c", subcore_axis_name="s")

    @functools.partial(
        pl.kernel,
        mesh=mesh,
        out_type=jax.ShapeDtypeStruct((NC, NPAD, D), jnp.float32),
        scratch_types=(
            [pltpu.VMEM((CHUNK,), jnp.int32) for _ in range(IR)]
            + [pltpu.VMEM((CHUNK,), jnp.int32) for _ in range(IR)]
            + [pltpu.VMEM((CHUNK, D), jnp.float32) for _ in range(RR)]
            + [pltpu.VMEM_SHARED((NPAD, D), jnp.float32)]
            + [pltpu.SemaphoreType.DMA for _ in range(IR + 2 * RR + 1)]
        ),
    )
    def seg_sum(h_hbm, src_hbm, dst_hbm, z_hbm, out_hbm, *scr):
        sidx = scr[:IR]
        didx = scr[IR:2 * IR]
        rows = scr[2 * IR:2 * IR + RR]
        acc = scr[2 * IR + RR]
        sems = scr[2 * IR + RR + 1:]
        isem = sems[:IR]
        gsem = sems[IR:IR + RR]
        ssem = sems[IR + RR:IR + 2 * RR]
        zsem = sems[IR + 2 * RR]
        cid = lax.axis_index("c")
        sid = lax.axis_index("s")
        wid = cid * NS + sid
        base = wid * EPW

        def fire_idx(c, i):
            off = base + c * CHUNK
            pltpu.async_copy(src_hbm.at[pl.ds(off, CHUNK)], sidx[i], isem[i])
            pltpu.async_copy(dst_hbm.at[pl.ds(off, CHUNK)], didx[i], isem[i])

        def wait_idx(i):
            pltpu.make_async_copy(
                src_hbm.at[pl.ds(base, CHUNK)], sidx[i], isem[i]).wait()
            pltpu.make_async_copy(
                dst_hbm.at[pl.ds(base, CHUNK)], didx[i], isem[i]).wait()

        zcp = pltpu.make_async_copy(
            z_hbm, acc.at[pl.ds(sid * ROWS_PER_TILE, ROWS_PER_TILE)], zsem)
        zcp.start()
        for c in range(3):
            fire_idx(c, c)
        for b in range(2):
            wait_idx(b)
            pltpu.async_copy(h_hbm.at[sidx[b]], rows[b], gsem[b])
        zcp.wait()
        plsc.subcore_barrier()

        def ring(k, carry):
            for u in range(IR):
                j = IR * k + u
                b = u % RR
                b2 = (u + 2) % RR
                i2 = (u + 2) % IR
                i3 = (u + 3) % IR

                @pl.when(j < NCHUNK)
                def _process():
                    pltpu.make_async_copy(
                        h_hbm.at[sidx[u]], rows[b], gsem[b]).wait()
                    pltpu.async_copy(
                        rows[b], acc.at[didx[u]], ssem[b], add=True)

                    @pl.when(j >= 2)
                    def _drain_s():
                        pltpu.make_async_copy(
                            rows[b2], acc.at[didx[i2]], ssem[b2]).wait()

                    @pl.when(j + 2 < NCHUNK)
                    def _fire_g():
                        wait_idx(i2)
                        pltpu.async_copy(
                            h_hbm.at[sidx[i2]], rows[b2], gsem[b2])

                    @pl.when(j + 3 < NCHUNK)
                    def _fire_i():
                        fire_idx(j + 3, i3)
            return carry
        lax.fori_loop(0, (NCHUNK + IR - 1) // IR, ring, 0)

        for c in (NCHUNK - 2, NCHUNK - 1):
            pltpu.make_async_copy(
                rows[c % RR], acc.at[didx[c % IR]], ssem[c % RR]).wait()
        plsc.subcore_barrier()

        pltpu.sync_copy(
            acc.at[pl.ds(sid * ROWS_PER_TILE, ROWS_PER_TILE)],
            out_hbm.at[cid, pl.ds(sid * ROWS_PER_TILE, ROWS_PER_TILE)])

    return seg_sum(h, src, dst, zrows)


def _dense_body(eps_ref, p2_ref, h_ref, w1_ref, b1_ref, g1_ref, be1_ref,
                w2_ref, b2_ref, g2_ref, be2_ref, hout_ref, sum_ref):
    hcur = h_ref[...]
    p2 = p2_ref[:, :N, :]
    pooled = p2[0] + p2[1] + (1.0 + eps_ref[0, 0]) * hcur
    z = jnp.dot(pooled, w1_ref[...], preferred_element_type=jnp.float32)
    z = z + b1_ref[...]
    mu = jnp.mean(z, axis=0, keepdims=True)
    var = jnp.mean((z - mu) ** 2, axis=0, keepdims=True)
    z = g1_ref[...] * (z - mu) * lax.rsqrt(var + 1e-5) + be1_ref[...]
    z = jnp.maximum(z, 0.0)
    z = jnp.dot(z, w2_ref[...], preferred_element_type=jnp.float32)
    z = z + b2_ref[...]
    mu2 = jnp.mean(z, axis=0, keepdims=True)
    var2 = jnp.mean((z - mu2) ** 2, axis=0, keepdims=True)
    z = g2_ref[...] * (z - mu2) * lax.rsqrt(var2 + 1e-5) + be2_ref[...]
    hn = jnp.maximum(z, 0.0)
    hout_ref[...] = hn
    sum_ref[...] = jnp.sum(hn, axis=0, keepdims=True)


def _dense_layer(parts, h, eps11, p, interpret=False):
    return pl.pallas_call(
        _dense_body,
        out_shape=(jax.ShapeDtypeStruct((N, D), jnp.float32),
                   jax.ShapeDtypeStruct((1, D), jnp.float32)),
        interpret=interpret,
    )(eps11, parts, h,
      p["W1"], p["b1"].reshape(1, D), p["g1"].reshape(1, D),
      p["be1"].reshape(1, D),
      p["W2"], p["b2"].reshape(1, D), p["g2"].reshape(1, D),
      p["be2"].reshape(1, D))


def _heads_body(x_ref, sums_ref, ws_ref, bs_ref, out_ref):
    acc = jnp.sum(bs_ref[...], axis=0)
    xsum = jnp.sum(x_ref[...], axis=0, keepdims=True)
    for l in range(NUM_LAYERS):
        s = xsum if l == 0 else sums_ref[pl.ds(l - 1, 1), :]
        acc = acc + jnp.dot(s, ws_ref[l],
                            preferred_element_type=jnp.float32)
    out_ref[...] = acc


def _heads(x, sums4, ws, bs, interpret=False):
    return pl.pallas_call(
        _heads_body,
        out_shape=jax.ShapeDtypeStruct((1, D), jnp.float32),
        interpret=interpret,
    )(x, sums4, ws, bs)


def kernel(x, edge_index, params):
    src = edge_index[0]
    dst = edge_index[1]
    zrows = jnp.zeros((ROWS_PER_TILE, D), jnp.float32)
    h = x
    sums = []
    for l in range(NUM_LAYERS - 1):
        parts = _sc_segment_sum(h, src, dst, zrows)
        h, s = _dense_layer(parts, h, params["eps"][l].reshape(1, 1),
                            params["layers"][l])
        sums.append(s)
    ws = jnp.stack([params["pred"][l]["W"] for l in range(NUM_LAYERS)])
    bs = jnp.stack([params["pred"][l]["b"].reshape(1, D)
                    for l in range(NUM_LAYERS)])
    sums4 = jnp.concatenate(sums, axis=0)
    return _heads(x, sums4, ws, bs)


# speedup vs baseline: 2.7526x; 1.0006x over previous
import functools

import jax
import jax.numpy as jnp
from jax import lax
from jax.experimental import pallas as pl
from jax.experimental.pallas import tpu as pltpu
from jax.experimental.pallas import tpu_sc as plsc

N = 10000
E = 320000
D = 128
NUM_LAYERS = 5

NC = 2
NS = 16
NW = NC * NS
EPW = E // NW
CHUNK = 40
NCHUNK = EPW // CHUNK
RR = 4
IR = 8
NPAD = 10240
ROWS_PER_TILE = NPAD // NS


def _sc_segment_sum(h, src, dst, zrows):
    mesh = plsc.VectorSubcoreMesh(core_axis_name="c", subcore_axis_name="s")

    @functools.partial(
        pl.kernel,
        mesh=mesh,
        out_type=jax.ShapeDtypeStruct((NC, NPAD, D), jnp.float32),
        scratch_types=(
            [pltpu.VMEM((CHUNK,), jnp.int32) for _ in range(IR)]
            + [pltpu.VMEM((CHUNK,), jnp.int32) for _ in range(IR)]
            + [pltpu.VMEM((CHUNK, D), jnp.float32) for _ in range(RR)]
            + [pltpu.VMEM_SHARED((NPAD, D), jnp.float32)]
            + [pltpu.SemaphoreType.DMA for _ in range(IR + 2 * RR + 1)]
        ),
    )
    def seg_sum(h_hbm, src_hbm, dst_hbm, z_hbm, out_hbm, *scr):
        sidx = scr[:IR]
        didx = scr[IR:2 * IR]
        rows = scr[2 * IR:2 * IR + RR]
        acc = scr[2 * IR + RR]
        sems = scr[2 * IR + RR + 1:]
        isem = sems[:IR]
        gsem = sems[IR:IR + RR]
        ssem = sems[IR + RR:IR + 2 * RR]
        zsem = sems[IR + 2 * RR]
        cid = lax.axis_index("c")
        sid = lax.axis_index("s")
        wid = cid * NS + sid
        base = wid * EPW

        def fire_idx(c, i):
            off = base + c * CHUNK
            pltpu.async_copy(src_hbm.at[pl.ds(off, CHUNK)], sidx[i], isem[i])
            pltpu.async_copy(dst_hbm.at[pl.ds(off, CHUNK)], didx[i], isem[i])

        def wait_idx(i):
            pltpu.make_async_copy(
                src_hbm.at[pl.ds(base, CHUNK)], sidx[i], isem[i]).wait()
            pltpu.make_async_copy(
                dst_hbm.at[pl.ds(base, CHUNK)], didx[i], isem[i]).wait()

        zcp = pltpu.make_async_copy(
            z_hbm, acc.at[pl.ds(sid * ROWS_PER_TILE, ROWS_PER_TILE)], zsem)
        zcp.start()
        for c in range(3):
            fire_idx(c, c)
        for b in range(2):
            wait_idx(b)
            pltpu.async_copy(h_hbm.at[sidx[b]], rows[b], gsem[b])
        zcp.wait()
        plsc.subcore_barrier()

        def ring(k, carry):
            for u in range(IR):
                j = IR * k + u
                b = u % RR
                b2 = (u + 2) % RR
                i2 = (u + 2) % IR
                i3 = (u + 3) % IR

                @pl.when(j < NCHUNK)
                def _process():
                    pltpu.make_async_copy(
                        h_hbm.at[sidx[u]], rows[b], gsem[b]).wait()
                    pltpu.async_copy(
                        rows[b], acc.at[didx[u]], ssem[b], add=True)

                    @pl.when(j >= 2)
                    def _drain_s():
                        pltpu.make_async_copy(
                            rows[b2], acc.at[didx[i2]], ssem[b2]).wait()

                    @pl.when(j + 2 < NCHUNK)
                    def _fire_g():
                        wait_idx(i2)
                        pltpu.async_copy(
                            h_hbm.at[sidx[i2]], rows[b2], gsem[b2])

                    @pl.when(j + 3 < NCHUNK)
                    def _fire_i():
                        fire_idx(j + 3, i3)
            return carry
        lax.fori_loop(0, (NCHUNK + IR - 1) // IR, ring, 0)

        for c in (NCHUNK - 2, NCHUNK - 1):
            pltpu.make_async_copy(
                rows[c % RR], acc.at[didx[c % IR]], ssem[c % RR]).wait()
        plsc.subcore_barrier()

        pltpu.sync_copy(
            acc.at[pl.ds(sid * ROWS_PER_TILE, ROWS_PER_TILE)],
            out_hbm.at[cid, pl.ds(sid * ROWS_PER_TILE, ROWS_PER_TILE)])

    return seg_sum(h, src, dst, zrows)


def _dense_body(eps_ref, p2_ref, h_ref, w1_ref, b1_ref, g1_ref, be1_ref,
                w2_ref, b2_ref, g2_ref, be2_ref, hout_ref, sum_ref):
    hcur = h_ref[...]
    p2 = p2_ref[:, :N, :]
    pooled = p2[0] + p2[1] + (1.0 + eps_ref[0, 0]) * hcur
    z = jnp.dot(pooled, w1_ref[...], preferred_element_type=jnp.float32)
    z = z + b1_ref[...]
    mu = jnp.mean(z, axis=0, keepdims=True)
    var = jnp.mean((z - mu) ** 2, axis=0, keepdims=True)
    z = g1_ref[...] * (z - mu) * lax.rsqrt(var + 1e-5) + be1_ref[...]
    z = jnp.maximum(z, 0.0)
    z = jnp.dot(z, w2_ref[...], preferred_element_type=jnp.float32)
    z = z + b2_ref[...]
    mu2 = jnp.mean(z, axis=0, keepdims=True)
    var2 = jnp.mean((z - mu2) ** 2, axis=0, keepdims=True)
    z = g2_ref[...] * (z - mu2) * lax.rsqrt(var2 + 1e-5) + be2_ref[...]
    hn = jnp.maximum(z, 0.0)
    hout_ref[...] = hn
    sum_ref[...] = jnp.sum(hn, axis=0, keepdims=True)


def _dense_layer(parts, h, eps11, p):
    return pl.pallas_call(
        _dense_body,
        out_shape=(jax.ShapeDtypeStruct((N, D), jnp.float32),
                   jax.ShapeDtypeStruct((1, D), jnp.float32)),
    )(eps11, parts, h,
      p["W1"], p["b1"].reshape(1, D), p["g1"].reshape(1, D),
      p["be1"].reshape(1, D),
      p["W2"], p["b2"].reshape(1, D), p["g2"].reshape(1, D),
      p["be2"].reshape(1, D))


def _heads_body(x_ref, sums_ref, ws_ref, bs_ref, out_ref):
    acc = jnp.sum(bs_ref[...], axis=0)
    xsum = jnp.sum(x_ref[...], axis=0, keepdims=True)
    for l in range(NUM_LAYERS):
        s = xsum if l == 0 else sums_ref[pl.ds(l - 1, 1), :]
        acc = acc + jnp.dot(s, ws_ref[l],
                            preferred_element_type=jnp.float32)
    out_ref[...] = acc


def _heads(x, sums4, ws, bs):
    return pl.pallas_call(
        _heads_body,
        out_shape=jax.ShapeDtypeStruct((1, D), jnp.float32),
    )(x, sums4, ws, bs)


def kernel(x, edge_index, params):
    src = edge_index[0]
    dst = edge_index[1]
    zrows = jnp.zeros((ROWS_PER_TILE, D), jnp.float32)
    h = x
    sums = []
    for l in range(NUM_LAYERS - 1):
        parts = _sc_segment_sum(h, src, dst, zrows)
        h, s = _dense_layer(parts, h, params["eps"][l].reshape(1, 1),
                            params["layers"][l])
        sums.append(s)
    ws = jnp.stack([params["pred"][l]["W"] for l in range(NUM_LAYERS)])
    bs = jnp.stack([params["pred"][l]["b"].reshape(1, D)
                    for l in range(NUM_LAYERS)])
    sums4 = jnp.concatenate(sums, axis=0)
    return _heads(x, sums4, ws, bs)
